# baseline (speedup 1.0000x reference)
import jax
import jax.numpy as jnp
from jax.experimental import pallas as pl
from jax.experimental.pallas import tpu as pltpu
from jax.experimental.pallas import tpu_sc as plsc

_FEAT = 8
_POS_DIM = 248
_ROW = _FEAT + _POS_DIM
_MAX_ROWS = 6
_NTAB = 7
_BLKB = 512


def _masked_slab_body(mid_ref, len_ref, tab_ref, wpos_ref, out_ref):
    n_m = out_ref.shape[0]
    n_pos = wpos_ref.shape[0]
    for c in range(n_m):
        mid = mid_ref[c]
        bid = jnp.minimum(mid, _NTAB - 1)
        tbl = tab_ref[pl.ds(bid, 1)]
        mrow = len_ref[bid] - 1
        vec8 = jnp.zeros((1, _FEAT), jnp.float32)
        for k in range(_MAX_ROWS):
            vec8 = vec8 + jnp.where(mrow == k, 1.0, 0.0) * tbl[0, k:k + 1, :]
        pos_row = wpos_ref[pl.ds(jnp.clip(mid, 0, n_pos - 1), 1), :]
        out_ref[c:c + 1, :] = jnp.concatenate([vec8, pos_row], axis=1)


def _unmasked_body(aid_ref, len_ref, data_ref, tab_ref, wnum_ref, wpos_ref,
                   out_un_ref, pos_s, emb_s, aux_s):
    blkb, n_un = data_ref.shape
    n_pos = wpos_ref.shape[0]

    @pl.when(pl.program_id(0) == 0)
    def _build_patterns():
        for c in range(n_un):
            aid = aid_ref[c]
            bid = jnp.minimum(aid, _NTAB)
            tid = jnp.minimum(bid, _NTAB - 1)
            pos_row = wpos_ref[pl.ds(jnp.clip(aid, 0, n_pos - 1), 1), :]
            pos_s[c:c + 1, :] = jnp.concatenate(
                [jnp.zeros((1, _FEAT), jnp.float32), pos_row], axis=1)
            tbl = tab_ref[pl.ds(tid, 1)]
            numflag = bid == _NTAB
            lanes = pl.ds(c * _FEAT, _FEAT)
            for k in range(_MAX_ROWS):
                emb_s[k:k + 1, lanes] = jnp.where(
                    numflag, jnp.zeros((1, _FEAT), jnp.float32),
                    tbl[0, k:k + 1, :])
            bound = jnp.where(numflag, -1, len_ref[tid] - 1)
            nrow1 = jnp.reshape(bound, (1, 1)).astype(jnp.float32)
            aux_s[0:1, lanes] = jnp.broadcast_to(nrow1, (1, _FEAT))
            aux_s[1:2, lanes] = wnum_ref[0:1, :]

    out_un_ref[...] = jnp.broadcast_to(pos_s[...][None], (blkb, n_un, _ROW))

    d96 = jnp.concatenate(
        [jnp.broadcast_to(data_ref[:, c:c + 1], (blkb, _FEAT))
         for c in range(n_un)], axis=1)
    di96 = jnp.clip(d96, 0.0, aux_s[0:1, :]).astype(jnp.int32)
    acc = d96 * aux_s[1:2, :]
    for k in range(_MAX_ROWS):
        acc = jnp.where(di96 == k, emb_s[k:k + 1, :], acc)
    emb3t = jnp.stack([acc[:, c * _FEAT:(c + 1) * _FEAT]
                       for c in range(n_un)], axis=0)
    emb3 = jnp.transpose(emb3t, (1, 0, 2))
    out_un_ref[:, :, 0:_FEAT] = emb3


def _replicate_masked(slab, bsz):
    n_m, row = slab.shape
    mesh = plsc.VectorSubcoreMesh(core_axis_name="core",
                                  subcore_axis_name="subcore")
    n_workers = mesh.num_cores * mesh.num_subcores
    seg = bsz // n_workers

    @pl.kernel(out_type=jax.ShapeDtypeStruct((bsz, n_m, row), slab.dtype),
               mesh=mesh,
               scratch_types=[pltpu.SemaphoreType.DMA])
    def repl(slab_hbm, o_hbm, sem):
        cid = jax.lax.axis_index("core")
        sid = jax.lax.axis_index("subcore")
        base = (cid * mesh.num_subcores + sid) * seg
        pltpu.async_copy(slab_hbm, o_hbm.at[base], sem).wait()
        n = 1
        while n < seg:
            pltpu.async_copy(o_hbm.at[pl.ds(base, n)],
                             o_hbm.at[pl.ds(base + n, n)], sem).wait()
            n *= 2

    return repl(slab)


def kernel(unmasked_data, unmasked_idx, masked_idx, W_Gender, W_Department,
           W_Grade, W_Extracurricular_Activities, W_Internet_Access_at_Home,
           W_Parent_Education_Level, W_Family_Income_Level, W_num, W_pos):
    tables = [W_Gender, W_Department, W_Grade, W_Extracurricular_Activities,
              W_Internet_Access_at_Home, W_Parent_Education_Level,
              W_Family_Income_Level]
    bsz, n_un = unmasked_data.shape
    n_m = masked_idx.shape[1]
    stacked = jnp.stack(
        [jnp.pad(t, ((0, _MAX_ROWS - t.shape[0]), (0, 0))) for t in tables])
    lens = jnp.array([t.shape[0] for t in tables], jnp.int32)
    aid = unmasked_idx[0, :]
    mid = masked_idx[0, :]

    m_slab = pl.pallas_call(
        _masked_slab_body,
        in_specs=[
            pl.BlockSpec(memory_space=pltpu.SMEM),
            pl.BlockSpec(memory_space=pltpu.SMEM),
            pl.BlockSpec((_NTAB, _MAX_ROWS, _FEAT), lambda: (0, 0, 0)),
            pl.BlockSpec(W_pos.shape, lambda: (0, 0)),
        ],
        out_specs=pl.BlockSpec((n_m, _ROW), lambda: (0, 0)),
        out_shape=jax.ShapeDtypeStruct((n_m, _ROW), jnp.float32),
    )(mid, lens, stacked, W_pos)

    out_m = _replicate_masked(m_slab, bsz)

    grid = (bsz // _BLKB,)
    out_un = pl.pallas_call(
        _unmasked_body,
        grid=grid,
        in_specs=[
            pl.BlockSpec(memory_space=pltpu.SMEM),
            pl.BlockSpec(memory_space=pltpu.SMEM),
            pl.BlockSpec((_BLKB, n_un), lambda i: (i, 0)),
            pl.BlockSpec((_NTAB, _MAX_ROWS, _FEAT), lambda i: (0, 0, 0)),
            pl.BlockSpec((1, _FEAT), lambda i: (0, 0)),
            pl.BlockSpec(W_pos.shape, lambda i: (0, 0)),
        ],
        out_specs=pl.BlockSpec((_BLKB, n_un, _ROW), lambda i: (i, 0, 0)),
        out_shape=jax.ShapeDtypeStruct((bsz, n_un, _ROW), jnp.float32),
        scratch_shapes=[
            pltpu.VMEM((12, _ROW), jnp.float32),
            pltpu.VMEM((_MAX_ROWS, 12 * _FEAT), jnp.float32),
            pltpu.VMEM((2, 12 * _FEAT), jnp.float32),
        ],
        compiler_params=pltpu.CompilerParams(
            dimension_semantics=("arbitrary",)),
    )(aid, lens, unmasked_data, stacked, W_num, W_pos)
    return out_un, out_m


# speedup vs baseline: 11.0195x; 11.0195x over previous
import jax
import jax.numpy as jnp
from jax.experimental import pallas as pl
from jax.experimental.pallas import tpu as pltpu

_FEAT = 8
_POS_DIM = 248
_ROW = _FEAT + _POS_DIM
_MAX_ROWS = 6
_NTAB = 7
_BLKB = 512


def _encode_body(aid_ref, mid_ref, len_ref,
                 data_ref, tab_ref, wnum_ref, wpos_ref,
                 out_un_ref, out_m_ref,
                 pos_s, emb_s, aux_s, m_s, bufu, bufm, semu, semm):
    blkb, n_un = data_ref.shape
    n_m = m_s.shape[0]
    n_pos = wpos_ref.shape[0]
    nsteps = pl.num_programs(0)
    i = pl.program_id(0)

    @pl.when(i == 0)
    def _build_patterns():
        for c in range(n_un):
            aid = aid_ref[c]
            bid = jnp.minimum(aid, _NTAB)
            tid = jnp.minimum(bid, _NTAB - 1)
            pos_row = wpos_ref[pl.ds(jnp.clip(aid, 0, n_pos - 1), 1), :]
            pos_s[c:c + 1, :] = jnp.concatenate(
                [jnp.zeros((1, _FEAT), jnp.float32), pos_row], axis=1)
            tbl = tab_ref[pl.ds(tid, 1)]
            numflag = bid == _NTAB
            lanes = pl.ds(c * _FEAT, _FEAT)
            for k in range(_MAX_ROWS):
                emb_s[k:k + 1, lanes] = jnp.where(
                    numflag, jnp.zeros((1, _FEAT), jnp.float32),
                    tbl[0, k:k + 1, :])
            bound = jnp.where(numflag, -1, len_ref[tid] - 1)
            nrow1 = jnp.reshape(bound, (1, 1)).astype(jnp.float32)
            aux_s[0:1, lanes] = jnp.broadcast_to(nrow1, (1, _FEAT))
            aux_s[1:2, lanes] = wnum_ref[0:1, :]
        for c in range(n_m):
            mid = mid_ref[c]
            bid = jnp.minimum(mid, _NTAB - 1)
            tbl = tab_ref[pl.ds(bid, 1)]
            mrow = len_ref[bid] - 1
            vec8 = jnp.zeros((1, _FEAT), jnp.float32)
            for k in range(_MAX_ROWS):
                vec8 = vec8 + jnp.where(mrow == k, 1.0, 0.0) * tbl[0, k:k + 1, :]
            pos_row = wpos_ref[pl.ds(jnp.clip(mid, 0, n_pos - 1), 1), :]
            m_s[c:c + 1, :] = jnp.concatenate([vec8, pos_row], axis=1)

    def _compute_into(bu, bm):
        bu[...] = jnp.broadcast_to(pos_s[...][None], (blkb, n_un, _ROW))
        bm[...] = jnp.broadcast_to(m_s[...][None], (blkb, n_m, _ROW))
        d96 = jnp.concatenate(
            [jnp.broadcast_to(data_ref[:, c:c + 1], (blkb, _FEAT))
             for c in range(n_un)], axis=1)
        di96 = jnp.clip(d96, 0.0, aux_s[0:1, :]).astype(jnp.int32)
        acc = d96 * aux_s[1:2, :]
        for k in range(_MAX_ROWS):
            acc = jnp.where(di96 == k, emb_s[k:k + 1, :], acc)
        emb3t = jnp.stack([acc[:, c * _FEAT:(c + 1) * _FEAT]
                           for c in range(n_un)], axis=0)
        bu[:, :, 0:_FEAT] = jnp.transpose(emb3t, (1, 0, 2))

    row_ds = pl.ds(i * blkb, blkb)

    for s in range(2):
        @pl.when(jax.lax.rem(i, 2) == s)
        def _slot(s=s):
            cp_u = pltpu.make_async_copy(bufu.at[s], out_un_ref.at[row_ds],
                                         semu.at[s])
            cp_m = pltpu.make_async_copy(bufm.at[s], out_m_ref.at[row_ds],
                                         semm.at[s])

            @pl.when(i >= 2)
            def _wait_prev():
                cp_u.wait()
                cp_m.wait()

            _compute_into(bufu.at[s], bufm.at[s])
            cp_u.start()
            cp_m.start()

    @pl.when(i == nsteps - 1)
    def _drain():
        for s in range(2):
            pltpu.make_async_copy(bufu.at[s], out_un_ref.at[row_ds],
                                  semu.at[s]).wait()
            pltpu.make_async_copy(bufm.at[s], out_m_ref.at[row_ds],
                                  semm.at[s]).wait()


def kernel(unmasked_data, unmasked_idx, masked_idx, W_Gender, W_Department,
           W_Grade, W_Extracurricular_Activities, W_Internet_Access_at_Home,
           W_Parent_Education_Level, W_Family_Income_Level, W_num, W_pos):
    tables = [W_Gender, W_Department, W_Grade, W_Extracurricular_Activities,
              W_Internet_Access_at_Home, W_Parent_Education_Level,
              W_Family_Income_Level]
    bsz, n_un = unmasked_data.shape
    n_m = masked_idx.shape[1]
    stacked = jnp.stack(
        [jnp.pad(t, ((0, _MAX_ROWS - t.shape[0]), (0, 0))) for t in tables])
    lens = jnp.array([t.shape[0] for t in tables], jnp.int32)
    aid = unmasked_idx[0, :]
    mid = masked_idx[0, :]

    grid = (bsz // _BLKB,)
    out_shapes = (
        jax.ShapeDtypeStruct((bsz, n_un, _ROW), jnp.float32),
        jax.ShapeDtypeStruct((bsz, n_m, _ROW), jnp.float32),
    )
    out_un, out_m = pl.pallas_call(
        _encode_body,
        grid=grid,
        in_specs=[
            pl.BlockSpec(memory_space=pltpu.SMEM),
            pl.BlockSpec(memory_space=pltpu.SMEM),
            pl.BlockSpec(memory_space=pltpu.SMEM),
            pl.BlockSpec((_BLKB, n_un), lambda i: (i, 0)),
            pl.BlockSpec((_NTAB, _MAX_ROWS, _FEAT), lambda i: (0, 0, 0)),
            pl.BlockSpec((1, _FEAT), lambda i: (0, 0)),
            pl.BlockSpec(W_pos.shape, lambda i: (0, 0)),
        ],
        out_specs=[
            pl.BlockSpec(memory_space=pl.ANY),
            pl.BlockSpec(memory_space=pl.ANY),
        ],
        out_shape=out_shapes,
        scratch_shapes=[
            pltpu.VMEM((12, _ROW), jnp.float32),
            pltpu.VMEM((_MAX_ROWS, 12 * _FEAT), jnp.float32),
            pltpu.VMEM((2, 12 * _FEAT), jnp.float32),
            pltpu.VMEM((6, _ROW), jnp.float32),
            pltpu.VMEM((2, _BLKB, 12, _ROW), jnp.float32),
            pltpu.VMEM((2, _BLKB, 6, _ROW), jnp.float32),
            pltpu.SemaphoreType.DMA((2,)),
            pltpu.SemaphoreType.DMA((2,)),
        ],
        compiler_params=pltpu.CompilerParams(
            dimension_semantics=("arbitrary",)),
    )(aid, mid, lens, unmasked_data, stacked, W_num, W_pos)
    return out_un, out_m


# speedup vs baseline: 11.0270x; 1.0007x over previous
import jax
import jax.numpy as jnp
from jax.experimental import pallas as pl
from jax.experimental.pallas import tpu as pltpu

_FEAT = 8
_POS_DIM = 248
_ROW = _FEAT + _POS_DIM
_MAX_ROWS = 6
_NTAB = 7
_BLKB = 512


def _encode_body(aid_ref, mid_ref, len_ref,
                 data_ref, tab_ref, wnum_ref, wpos_ref,
                 out_un_ref, out_m_ref,
                 pos_s, emb_s, aux_s, m_s, bufu, bufm, semu, semm):
    blkb, n_un = data_ref.shape
    n_m = m_s.shape[0]
    n_pos = wpos_ref.shape[0]
    nsteps = pl.num_programs(0)
    i = pl.program_id(0)

    @pl.when(i == 0)
    def _build_patterns():
        for c in range(n_un):
            aid = aid_ref[c]
            bid = jnp.minimum(aid, _NTAB)
            tid = jnp.minimum(bid, _NTAB - 1)
            pos_row = wpos_ref[pl.ds(jnp.clip(aid, 0, n_pos - 1), 1), :]
            pos_s[c:c + 1, :] = jnp.concatenate(
                [jnp.zeros((1, _FEAT), jnp.float32), pos_row], axis=1)
            tbl = tab_ref[pl.ds(tid, 1)]
            numflag = bid == _NTAB
            lanes = pl.ds(c * _FEAT, _FEAT)
            for k in range(_MAX_ROWS):
                emb_s[k:k + 1, lanes] = jnp.where(
                    numflag, jnp.zeros((1, _FEAT), jnp.float32),
                    tbl[0, k:k + 1, :])
            bound = jnp.where(numflag, -1, len_ref[tid] - 1)
            nrow1 = jnp.reshape(bound, (1, 1)).astype(jnp.float32)
            aux_s[0:1, lanes] = jnp.broadcast_to(nrow1, (1, _FEAT))
            aux_s[1:2, lanes] = wnum_ref[0:1, :]
        for c in range(n_m):
            mid = mid_ref[c]
            bid = jnp.minimum(mid, _NTAB - 1)
            tbl = tab_ref[pl.ds(bid, 1)]
            mrow = len_ref[bid] - 1
            vec8 = jnp.zeros((1, _FEAT), jnp.float32)
            for k in range(_MAX_ROWS):
                vec8 = vec8 + jnp.where(mrow == k, 1.0, 0.0) * tbl[0, k:k + 1, :]
            pos_row = wpos_ref[pl.ds(jnp.clip(mid, 0, n_pos - 1), 1), :]
            m_s[c:c + 1, :] = jnp.concatenate([vec8, pos_row], axis=1)
        for s in range(2):
            bufu[s] = jnp.broadcast_to(pos_s[...][None], (blkb, n_un, _ROW))
        bufm[...] = jnp.broadcast_to(m_s[...][None], (blkb, n_m, _ROW))

    def _compute_into(bu):
        d96 = jnp.concatenate(
            [jnp.broadcast_to(data_ref[:, c:c + 1], (blkb, _FEAT))
             for c in range(n_un)], axis=1)
        di96 = jnp.clip(d96, 0.0, aux_s[0:1, :]).astype(jnp.int32)
        acc = d96 * aux_s[1:2, :]
        for k in range(_MAX_ROWS):
            acc = jnp.where(di96 == k, emb_s[k:k + 1, :], acc)
        emb3t = jnp.stack([acc[:, c * _FEAT:(c + 1) * _FEAT]
                           for c in range(n_un)], axis=0)
        bu[:, :, 0:_FEAT] = jnp.transpose(emb3t, (1, 0, 2))

    row_ds = pl.ds(i * blkb, blkb)

    for s in range(2):
        @pl.when(jax.lax.rem(i, 2) == s)
        def _slot(s=s):
            cp_u = pltpu.make_async_copy(bufu.at[s], out_un_ref.at[row_ds],
                                         semu.at[s])
            cp_m = pltpu.make_async_copy(bufm, out_m_ref.at[row_ds],
                                         semm.at[s])

            @pl.when(i >= 2)
            def _wait_prev():
                cp_u.wait()
                cp_m.wait()

            _compute_into(bufu.at[s])
            cp_u.start()
            cp_m.start()

    @pl.when(i == nsteps - 1)
    def _drain():
        for s in range(2):
            pltpu.make_async_copy(bufu.at[s], out_un_ref.at[row_ds],
                                  semu.at[s]).wait()
            pltpu.make_async_copy(bufm, out_m_ref.at[row_ds],
                                  semm.at[s]).wait()


def kernel(unmasked_data, unmasked_idx, masked_idx, W_Gender, W_Department,
           W_Grade, W_Extracurricular_Activities, W_Internet_Access_at_Home,
           W_Parent_Education_Level, W_Family_Income_Level, W_num, W_pos):
    tables = [W_Gender, W_Department, W_Grade, W_Extracurricular_Activities,
              W_Internet_Access_at_Home, W_Parent_Education_Level,
              W_Family_Income_Level]
    bsz, n_un = unmasked_data.shape
    n_m = masked_idx.shape[1]
    stacked = jnp.stack(
        [jnp.pad(t, ((0, _MAX_ROWS - t.shape[0]), (0, 0))) for t in tables])
    lens = jnp.array([t.shape[0] for t in tables], jnp.int32)
    aid = unmasked_idx[0, :]
    mid = masked_idx[0, :]

    grid = (bsz // _BLKB,)
    out_shapes = (
        jax.ShapeDtypeStruct((bsz, n_un, _ROW), jnp.float32),
        jax.ShapeDtypeStruct((bsz, n_m, _ROW), jnp.float32),
    )
    out_un, out_m = pl.pallas_call(
        _encode_body,
        grid=grid,
        in_specs=[
            pl.BlockSpec(memory_space=pltpu.SMEM),
            pl.BlockSpec(memory_space=pltpu.SMEM),
            pl.BlockSpec(memory_space=pltpu.SMEM),
            pl.BlockSpec((_BLKB, n_un), lambda i: (i, 0)),
            pl.BlockSpec((_NTAB, _MAX_ROWS, _FEAT), lambda i: (0, 0, 0)),
            pl.BlockSpec((1, _FEAT), lambda i: (0, 0)),
            pl.BlockSpec(W_pos.shape, lambda i: (0, 0)),
        ],
        out_specs=[
            pl.BlockSpec(memory_space=pl.ANY),
            pl.BlockSpec(memory_space=pl.ANY),
        ],
        out_shape=out_shapes,
        scratch_shapes=[
            pltpu.VMEM((12, _ROW), jnp.float32),
            pltpu.VMEM((_MAX_ROWS, 12 * _FEAT), jnp.float32),
            pltpu.VMEM((2, 12 * _FEAT), jnp.float32),
            pltpu.VMEM((6, _ROW), jnp.float32),
            pltpu.VMEM((2, _BLKB, 12, _ROW), jnp.float32),
            pltpu.VMEM((_BLKB, 6, _ROW), jnp.float32),
            pltpu.SemaphoreType.DMA((2,)),
            pltpu.SemaphoreType.DMA((2,)),
        ],
        compiler_params=pltpu.CompilerParams(
            dimension_semantics=("arbitrary",)),
    )(aid, mid, lens, unmasked_data, stacked, W_num, W_pos)
    return out_un, out_m
